# baseline (speedup 1.0000x reference)
import functools

import jax
import jax.numpy as jnp
from jax import lax
from jax.experimental import pallas as pl
from jax.experimental.pallas import tpu as pltpu
from jax.experimental.pallas import tpu_sc as plsc

_EPS = 1e-5

_D = 1024
_ROWS = 32768
_NC = 2
_NS = 16
_NW = _NC * _NS
_RPW = _ROWS // _NW
_C = 32
_CHUNKS = _RPW // _C
_LANES = _D // 16


def _newton_rsqrt(v):
    i = lax.bitcast_convert_type(v, jnp.int32)
    i = jnp.int32(0x5F3759DF) - lax.shift_right_logical(i, 1)
    y = lax.bitcast_convert_type(i, jnp.float32)
    for _ in range(3):
        y = y * (1.5 - 0.5 * v * y * y)
    return y


def _sc_body(src, maskf, gamma, beta, out, buf, mbuf, gbuf, bbuf):
    c = lax.axis_index("c")
    s = lax.axis_index("s")
    wid = s * _NC + c
    pltpu.sync_copy(gamma, gbuf)
    pltpu.sync_copy(beta, bbuf)
    base = wid * _RPW

    def chunk_body(k, carry):
        row0 = base + k * _C
        pltpu.sync_copy(src.at[pl.ds(row0 * _D, _C * _D)], buf)
        pltpu.sync_copy(maskf.at[pl.ds(row0, _C)], mbuf)

        def row_body(r, carry2):
            off = r * _D

            def p1(j, acc):
                return acc + buf[pl.ds(off + j * 16, 16)]

            acc = lax.fori_loop(0, _LANES, p1, jnp.zeros((16,), jnp.float32))
            mean = jnp.sum(acc) * (1.0 / _D)

            def p2(j, acc2):
                x = buf[pl.ds(off + j * 16, 16)]
                d = x - mean
                return acc2 + d * d

            acc2 = lax.fori_loop(0, _LANES, p2, jnp.zeros((16,), jnp.float32))
            var = jnp.sum(acc2) * (1.0 / _D)
            inv = _newton_rsqrt(jnp.full((16,), var + _EPS, jnp.float32))
            m = plsc.load_gather(mbuf, [jnp.full((16,), r, jnp.int32)])
            sel = m > 0.5

            def p3(j, _):
                sl = pl.ds(off + j * 16, 16)
                gl = pl.ds(j * 16, 16)
                x = buf[sl]
                y = (x - mean) * inv * gbuf[gl] + bbuf[gl]
                buf[sl] = jnp.where(sel, y, x)
                return 0

            lax.fori_loop(0, _LANES, p3, 0)
            return 0

        lax.fori_loop(0, _C, row_body, 0)
        pltpu.sync_copy(buf, out.at[pl.ds(row0 * _D, _C * _D)])
        return 0

    lax.fori_loop(0, _CHUNKS, chunk_body, 0)


def kernel(src, gamma, beta, padding_mask):
    b, n, d = src.shape
    rows = b * n
    x = src.reshape(rows * d)
    maskf = padding_mask.reshape(rows).astype(jnp.float32)
    sc_call = functools.partial(
        pl.kernel,
        mesh=plsc.VectorSubcoreMesh(core_axis_name="c", subcore_axis_name="s"),
        out_type=jax.ShapeDtypeStruct((rows * d,), jnp.float32),
        scratch_types=[
            pltpu.VMEM((_C * _D,), jnp.float32),
            pltpu.VMEM((_C,), jnp.float32),
            pltpu.VMEM((_D,), jnp.float32),
            pltpu.VMEM((_D,), jnp.float32),
        ],
        compiler_params=pltpu.CompilerParams(needs_layout_passes=False),
    )(_sc_body)
    out = sc_call(x, maskf, gamma, beta)
    return out.reshape(b, n, d)


# speedup vs baseline: 2.0888x; 2.0888x over previous
import functools

import jax
import jax.numpy as jnp
from jax import lax
from jax.experimental import pallas as pl
from jax.experimental.pallas import tpu as pltpu
from jax.experimental.pallas import tpu_sc as plsc

_EPS = 1e-5

_D = 1024
_ROWS = 32768
_NC = 2
_NS = 16
_NW = _NC * _NS
_RPW = _ROWS // _NW
_C = 16
_CHUNKS = _RPW // _C
_NBUF = 4
_LANES = _D // 16


def _newton_rsqrt(v):
    i = lax.bitcast_convert_type(v, jnp.int32)
    i = jnp.int32(0x5F3759DF) - lax.shift_right_logical(i, 1)
    y = lax.bitcast_convert_type(i, jnp.float32)
    for _ in range(3):
        y = y * (1.5 - 0.5 * v * y * y)
    return y


def _sc_body(src, maskf, gamma, beta, out, b0, b1, b2, b3, mbuf, gbuf, bbuf,
             si0, si1, si2, si3, so0, so1, so2, so3):
    bufs = [b0, b1, b2, b3]
    sin = [si0, si1, si2, si3]
    sout = [so0, so1, so2, so3]
    c = lax.axis_index("c")
    s = lax.axis_index("s")
    wid = s * _NC + c
    base = wid * _RPW
    pltpu.sync_copy(gamma, gbuf)
    pltpu.sync_copy(beta, bbuf)
    pltpu.sync_copy(maskf.at[pl.ds(base, _RPW)], mbuf)

    def in_slice(k):
        return src.at[pl.ds((base + k * _C) * _D, _C * _D)]

    def out_slice(k):
        return out.at[pl.ds((base + k * _C) * _D, _C * _D)]

    pltpu.async_copy(in_slice(0), bufs[0], sin[0])
    pltpu.async_copy(in_slice(1), bufs[1], sin[1])

    def compute_chunk(k, buf):
        def row_body(r, carry):
            off = r * _D

            def p1(j, acc):
                s1, s2 = acc
                x = buf[pl.ds(off + j * 16, 16)]
                return (s1 + x, s2 + x * x)

            zero = jnp.zeros((16,), jnp.float32)
            s1, s2 = lax.fori_loop(0, _LANES, p1, (zero, zero), unroll=True)
            mean = jnp.sum(s1) * (1.0 / _D)
            var = jnp.maximum(jnp.sum(s2) * (1.0 / _D) - mean * mean, 0.0)
            inv = _newton_rsqrt(jnp.full((16,), var + _EPS, jnp.float32))
            m = plsc.load_gather(mbuf, [jnp.full((16,), k * _C + r, jnp.int32)])
            sel = m > 0.5

            def p2(j, _):
                sl = pl.ds(off + j * 16, 16)
                gl = pl.ds(j * 16, 16)
                x = buf[sl]
                y = (x - mean) * inv * gbuf[gl] + bbuf[gl]
                buf[sl] = jnp.where(sel, y, x)
                return 0

            lax.fori_loop(0, _LANES, p2, 0, unroll=True)
            return 0

        lax.fori_loop(0, _C, row_body, 0)

    def outer(i, carry):
        for b in range(_NBUF):
            k = i * _NBUF + b
            nb = (b + 2) % _NBUF

            @pl.when(k >= 2)
            def _():
                pltpu.make_async_copy(bufs[nb], out_slice(k - 2),
                                      sout[nb]).wait()

            @pl.when(k < _CHUNKS - 2)
            def _():
                pltpu.async_copy(in_slice(k + 2), bufs[nb], sin[nb])

            pltpu.make_async_copy(in_slice(k), bufs[b], sin[b]).wait()
            compute_chunk(k, bufs[b])
            pltpu.async_copy(bufs[b], out_slice(k), sout[b])
        return 0

    lax.fori_loop(0, _CHUNKS // _NBUF, outer, 0)
    pltpu.make_async_copy(bufs[(_CHUNKS - 2) % _NBUF],
                          out_slice(_CHUNKS - 2),
                          sout[(_CHUNKS - 2) % _NBUF]).wait()
    pltpu.make_async_copy(bufs[(_CHUNKS - 1) % _NBUF],
                          out_slice(_CHUNKS - 1),
                          sout[(_CHUNKS - 1) % _NBUF]).wait()


def kernel(src, gamma, beta, padding_mask):
    b, n, d = src.shape
    rows = b * n
    x = src.reshape(rows * d)
    maskf = padding_mask.reshape(rows).astype(jnp.float32)
    sc_call = functools.partial(
        pl.kernel,
        mesh=plsc.VectorSubcoreMesh(core_axis_name="c", subcore_axis_name="s"),
        out_type=jax.ShapeDtypeStruct((rows * d,), jnp.float32),
        scratch_types=[
            pltpu.VMEM((_C * _D,), jnp.float32),
            pltpu.VMEM((_C * _D,), jnp.float32),
            pltpu.VMEM((_C * _D,), jnp.float32),
            pltpu.VMEM((_C * _D,), jnp.float32),
            pltpu.VMEM((_RPW,), jnp.float32),
            pltpu.VMEM((_D,), jnp.float32),
            pltpu.VMEM((_D,), jnp.float32),
            pltpu.SemaphoreType.DMA,
            pltpu.SemaphoreType.DMA,
            pltpu.SemaphoreType.DMA,
            pltpu.SemaphoreType.DMA,
            pltpu.SemaphoreType.DMA,
            pltpu.SemaphoreType.DMA,
            pltpu.SemaphoreType.DMA,
            pltpu.SemaphoreType.DMA,
        ],
        compiler_params=pltpu.CompilerParams(needs_layout_passes=False),
    )(_sc_body)
    out = sc_call(x, maskf, gamma, beta)
    return out.reshape(b, n, d)


# speedup vs baseline: 2.5171x; 1.2051x over previous
import functools

import jax
import jax.numpy as jnp
from jax import lax
from jax.experimental import pallas as pl
from jax.experimental.pallas import tpu as pltpu
from jax.experimental.pallas import tpu_sc as plsc

_EPS = 1e-5

_D = 1024
_ROWS = 32768
_NC = 2
_NS = 16
_NW = _NC * _NS
_RPW = _ROWS // _NW
_C = 16
_CHUNKS = _RPW // _C
_NBUF = 4
_LANES = _D // 16


def _newton_rsqrt(v):
    i = lax.bitcast_convert_type(v, jnp.int32)
    i = jnp.int32(0x5F3759DF) - lax.shift_right_logical(i, 1)
    y = lax.bitcast_convert_type(i, jnp.float32)
    for _ in range(3):
        y = y * (1.5 - 0.5 * v * y * y)
    return y


def _sc_body(src, maskf, gamma, beta, out, b0, b1, b2, b3, mbuf, gbuf, bbuf,
             si0, si1, si2, si3, so0, so1, so2, so3):
    bufs = [b0, b1, b2, b3]
    sin = [si0, si1, si2, si3]
    sout = [so0, so1, so2, so3]
    c = lax.axis_index("c")
    s = lax.axis_index("s")
    wid = s * _NC + c
    base = wid * _RPW
    pltpu.sync_copy(gamma, gbuf)
    pltpu.sync_copy(beta, bbuf)
    pltpu.sync_copy(maskf.at[pl.ds(base, _RPW)], mbuf)

    def in_slice(k):
        return src.at[pl.ds(base + k * _C, _C)]

    def out_slice(k):
        return out.at[pl.ds(base + k * _C, _C)]

    pltpu.async_copy(in_slice(0), bufs[0], sin[0])
    pltpu.async_copy(in_slice(1), bufs[1], sin[1])

    def compute_chunk(k, buf):
        def row_body(r, carry):
            def p1(j, acc):
                s1, s2 = acc
                x = buf[r, pl.ds(j * 16, 16)]
                return (s1 + x, s2 + x * x)

            zero = jnp.zeros((16,), jnp.float32)
            s1, s2 = lax.fori_loop(0, _LANES, p1, (zero, zero), unroll=True)
            mean = jnp.sum(s1) * (1.0 / _D)
            var = jnp.maximum(jnp.sum(s2) * (1.0 / _D) - mean * mean, 0.0)
            inv = _newton_rsqrt(jnp.full((16,), var + _EPS, jnp.float32))
            m = plsc.load_gather(mbuf, [jnp.full((16,), k * _C + r, jnp.int32)])
            sel = m > 0.5

            def p2(j, _):
                gl = pl.ds(j * 16, 16)
                x = buf[r, gl]
                y = (x - mean) * inv * gbuf[gl] + bbuf[gl]
                buf[r, gl] = jnp.where(sel, y, x)
                return 0

            lax.fori_loop(0, _LANES, p2, 0, unroll=True)
            return 0

        lax.fori_loop(0, _C, row_body, 0)

    def outer(i, carry):
        for b in range(_NBUF):
            k = i * _NBUF + b
            nb = (b + 2) % _NBUF

            @pl.when(k >= 2)
            def _():
                pltpu.make_async_copy(bufs[nb], out_slice(k - 2),
                                      sout[nb]).wait()

            @pl.when(k < _CHUNKS - 2)
            def _():
                pltpu.async_copy(in_slice(k + 2), bufs[nb], sin[nb])

            pltpu.make_async_copy(in_slice(k), bufs[b], sin[b]).wait()
            compute_chunk(k, bufs[b])
            pltpu.async_copy(bufs[b], out_slice(k), sout[b])
        return 0

    lax.fori_loop(0, _CHUNKS // _NBUF, outer, 0)
    pltpu.make_async_copy(bufs[(_CHUNKS - 2) % _NBUF],
                          out_slice(_CHUNKS - 2),
                          sout[(_CHUNKS - 2) % _NBUF]).wait()
    pltpu.make_async_copy(bufs[(_CHUNKS - 1) % _NBUF],
                          out_slice(_CHUNKS - 1),
                          sout[(_CHUNKS - 1) % _NBUF]).wait()


def kernel(src, gamma, beta, padding_mask):
    b, n, d = src.shape
    rows = b * n
    x = src.reshape(rows, d)
    maskf = padding_mask.reshape(rows).astype(jnp.float32)
    sc_call = functools.partial(
        pl.kernel,
        mesh=plsc.VectorSubcoreMesh(core_axis_name="c", subcore_axis_name="s"),
        out_type=jax.ShapeDtypeStruct((rows, d), jnp.float32),
        scratch_types=[
            pltpu.VMEM((_C, _D), jnp.float32),
            pltpu.VMEM((_C, _D), jnp.float32),
            pltpu.VMEM((_C, _D), jnp.float32),
            pltpu.VMEM((_C, _D), jnp.float32),
            pltpu.VMEM((_RPW,), jnp.float32),
            pltpu.VMEM((_D,), jnp.float32),
            pltpu.VMEM((_D,), jnp.float32),
            pltpu.SemaphoreType.DMA,
            pltpu.SemaphoreType.DMA,
            pltpu.SemaphoreType.DMA,
            pltpu.SemaphoreType.DMA,
            pltpu.SemaphoreType.DMA,
            pltpu.SemaphoreType.DMA,
            pltpu.SemaphoreType.DMA,
            pltpu.SemaphoreType.DMA,
        ],
        compiler_params=pltpu.CompilerParams(needs_layout_passes=False),
    )(_sc_body)
    out = sc_call(x, maskf, gamma, beta)
    return out.reshape(b, n, d)


# speedup vs baseline: 2.6225x; 1.0419x over previous
import functools

import jax
import jax.numpy as jnp
from jax import lax
from jax.experimental import pallas as pl
from jax.experimental.pallas import tpu as pltpu
from jax.experimental.pallas import tpu_sc as plsc

_EPS = 1e-5

_D = 1024
_ROWS = 32768
_NC = 2
_NS = 16
_NW = _NC * _NS
_RPW = _ROWS // _NW
_C = 16
_CHUNKS = _RPW // _C
_NBUF = 4
_LANES = _D // 16


def _newton_rsqrt(v):
    i = lax.bitcast_convert_type(v, jnp.int32)
    i = jnp.int32(0x5F3759DF) - lax.shift_right_logical(i, 1)
    y = lax.bitcast_convert_type(i, jnp.float32)
    for _ in range(3):
        y = y * (1.5 - 0.5 * v * y * y)
    return y


def _sc_body(src, maskf, gamma, beta, out, b0, b1, b2, b3, mbuf, gbuf, bbuf,
             si0, si1, si2, si3, so0, so1, so2, so3):
    bufs = [b0, b1, b2, b3]
    sin = [si0, si1, si2, si3]
    sout = [so0, so1, so2, so3]
    c = lax.axis_index("c")
    s = lax.axis_index("s")
    wid = s * _NC + c
    base = wid * _RPW
    pltpu.sync_copy(gamma, gbuf)
    pltpu.sync_copy(beta, bbuf)
    pltpu.sync_copy(maskf.at[pl.ds(base, _RPW)], mbuf)

    def in_slice(k):
        return src.at[pl.ds(base + k * _C, _C)]

    def out_slice(k):
        return out.at[pl.ds(base + k * _C, _C)]

    pltpu.async_copy(in_slice(0), bufs[0], sin[0])
    pltpu.async_copy(in_slice(1), bufs[1], sin[1])

    def compute_chunk(k, buf):
        def row_body(r, carry):
            zero = jnp.zeros((16,), jnp.float32)
            s1 = [zero] * 4
            s2 = [zero] * 4
            for j in range(_LANES):
                x = buf[r, pl.ds(j * 16, 16)]
                t = j % 4
                s1[t] = s1[t] + x
                s2[t] = s2[t] + x * x
            s1 = (s1[0] + s1[1]) + (s1[2] + s1[3])
            s2 = (s2[0] + s2[1]) + (s2[2] + s2[3])
            mean = jnp.sum(s1) * (1.0 / _D)
            var = jnp.maximum(jnp.sum(s2) * (1.0 / _D) - mean * mean, 0.0)
            inv = _newton_rsqrt(jnp.full((16,), var + _EPS, jnp.float32))
            m = plsc.load_gather(mbuf, [jnp.full((16,), k * _C + r, jnp.int32)])
            sel = m > 0.5

            def p2(j, _):
                gl = pl.ds(j * 16, 16)
                x = buf[r, gl]
                y = (x - mean) * inv * gbuf[gl] + bbuf[gl]
                buf[r, gl] = jnp.where(sel, y, x)
                return 0

            lax.fori_loop(0, _LANES, p2, 0, unroll=True)
            return 0

        lax.fori_loop(0, _C, row_body, 0)

    def outer(i, carry):
        for b in range(_NBUF):
            k = i * _NBUF + b
            nb = (b + 2) % _NBUF

            @pl.when(k >= 2)
            def _():
                pltpu.make_async_copy(bufs[nb], out_slice(k - 2),
                                      sout[nb]).wait()

            @pl.when(k < _CHUNKS - 2)
            def _():
                pltpu.async_copy(in_slice(k + 2), bufs[nb], sin[nb])

            pltpu.make_async_copy(in_slice(k), bufs[b], sin[b]).wait()
            compute_chunk(k, bufs[b])
            pltpu.async_copy(bufs[b], out_slice(k), sout[b])
        return 0

    lax.fori_loop(0, _CHUNKS // _NBUF, outer, 0)
    pltpu.make_async_copy(bufs[(_CHUNKS - 2) % _NBUF],
                          out_slice(_CHUNKS - 2),
                          sout[(_CHUNKS - 2) % _NBUF]).wait()
    pltpu.make_async_copy(bufs[(_CHUNKS - 1) % _NBUF],
                          out_slice(_CHUNKS - 1),
                          sout[(_CHUNKS - 1) % _NBUF]).wait()


def kernel(src, gamma, beta, padding_mask):
    b, n, d = src.shape
    rows = b * n
    x = src.reshape(rows, d)
    maskf = padding_mask.reshape(rows).astype(jnp.float32)
    sc_call = functools.partial(
        pl.kernel,
        mesh=plsc.VectorSubcoreMesh(core_axis_name="c", subcore_axis_name="s"),
        out_type=jax.ShapeDtypeStruct((rows, d), jnp.float32),
        scratch_types=[
            pltpu.VMEM((_C, _D), jnp.float32),
            pltpu.VMEM((_C, _D), jnp.float32),
            pltpu.VMEM((_C, _D), jnp.float32),
            pltpu.VMEM((_C, _D), jnp.float32),
            pltpu.VMEM((_RPW,), jnp.float32),
            pltpu.VMEM((_D,), jnp.float32),
            pltpu.VMEM((_D,), jnp.float32),
            pltpu.SemaphoreType.DMA,
            pltpu.SemaphoreType.DMA,
            pltpu.SemaphoreType.DMA,
            pltpu.SemaphoreType.DMA,
            pltpu.SemaphoreType.DMA,
            pltpu.SemaphoreType.DMA,
            pltpu.SemaphoreType.DMA,
            pltpu.SemaphoreType.DMA,
        ],
        compiler_params=pltpu.CompilerParams(needs_layout_passes=False),
    )(_sc_body)
    out = sc_call(x, maskf, gamma, beta)
    return out.reshape(b, n, d)


# speedup vs baseline: 14.7433x; 5.6219x over previous
import jax
import jax.numpy as jnp
from jax.experimental import pallas as pl
from jax.experimental.pallas import tpu as pltpu

_EPS = 1e-5
_BLOCK_ROWS = 2048


def _masked_ln_kernel(x_ref, m_ref, g_ref, b_ref, o_ref):
    x = x_ref[...]
    mean = jnp.mean(x, axis=1, keepdims=True)
    c = x - mean
    var = jnp.mean(c * c, axis=1, keepdims=True)
    y = c * jax.lax.rsqrt(var + _EPS) * g_ref[...] + b_ref[...]
    o_ref[...] = jnp.where(m_ref[...], y, x)


def kernel(src, gamma, beta, padding_mask):
    b, n, d = src.shape
    rows = b * n
    block = _BLOCK_ROWS
    x = src.reshape(rows, d)
    m = padding_mask.reshape(rows, 1)
    g = gamma.reshape(1, d)
    bt = beta.reshape(1, d)
    out = pl.pallas_call(
        _masked_ln_kernel,
        grid=(rows // block,),
        in_specs=[
            pl.BlockSpec((block, d), lambda i: (i, 0)),
            pl.BlockSpec((block, 1), lambda i: (i, 0)),
            pl.BlockSpec((1, d), lambda i: (0, 0)),
            pl.BlockSpec((1, d), lambda i: (0, 0)),
        ],
        out_specs=pl.BlockSpec((block, d), lambda i: (i, 0)),
        out_shape=jax.ShapeDtypeStruct((rows, d), src.dtype),
        compiler_params=pltpu.CompilerParams(
            dimension_semantics=("parallel",),
        ),
    )(x, m, g, bt)
    return out.reshape(b, n, d)
